# baseline (speedup 1.0000x reference)
import functools

import jax
import jax.numpy as jnp
from jax import lax
from jax.experimental import pallas as pl
from jax.experimental.pallas import tpu as pltpu
from jax.experimental.pallas import tpu_sc as plsc

_NC = 2
_NS = 16
_NW = _NC * _NS

_MESH = plsc.VectorSubcoreMesh(
    core_axis_name="c", subcore_axis_name="s", num_cores=_NC, num_subcores=_NS
)

_CH = 128

_NZW = 10


def _zero_vmem(buf, n_rows, width):
    z16 = jnp.zeros((16,), jnp.float32)

    def body(i, c):
        for j in range(width // 16):
            buf[i, pl.ds(16 * j, 16)] = z16
        return c

    lax.fori_loop(0, n_rows, body, 0)


def _zero_rows(zstage, acc, row0, n_rows):
    n_full, tail = divmod(n_rows, _CH)
    assert tail % 8 == 0

    def body(k, c):
        pltpu.sync_copy(zstage, acc.at[pl.ds(row0 + k * _CH, _CH)])
        return c

    lax.fori_loop(0, n_full, body, 0)
    if tail:
        pltpu.sync_copy(
            zstage.at[pl.ds(0, tail)], acc.at[pl.ds(row0 + n_full * _CH, tail)]
        )


def _copy_out(acc, out_dst, row0, n_rows):
    pltpu.sync_copy(acc.at[pl.ds(row0, n_rows)], out_dst.at[pl.ds(row0, n_rows)])


def _make_degree(N, E_pad):
    NCK = E_pad // (_NW * _CH)
    assert NCK * _NW * _CH == E_pad and NCK % 8 == 0
    ROWS_Z = N // _NZW
    assert ROWS_Z * _NZW == N and ROWS_Z % 8 == 0
    GRP = 8

    @functools.partial(
        pl.kernel,
        out_type=jax.ShapeDtypeStruct((_NC, N, 16), jnp.float32),
        mesh=_MESH,
        scratch_types=[
            pltpu.VMEM((NCK, _CH), jnp.int32),
            pltpu.VMEM((_CH, 16), jnp.float32),
            pltpu.VMEM((_CH, 16), jnp.float32),
            pltpu.VMEM_SHARED((N + _JUNK, 16), jnp.float32),
            pltpu.SemaphoreType.DMA,
        ],
    )
    def deg_kernel(dst_hbm, out_hbm, didx, ones, zbuf, acc, sem):
        cid = lax.axis_index("c")
        sid = lax.axis_index("s")
        wid = cid * _NS + sid
        one16 = jnp.ones((16,), jnp.float32)
        z16 = jnp.zeros((16,), jnp.float32)

        def init_body(i, c):
            ones[i, pl.ds(0, 16)] = one16
            zbuf[i, pl.ds(0, 16)] = z16
            return c

        lax.fori_loop(0, _CH, init_body, 0)
        pltpu.sync_copy(dst_hbm.at[pl.ds(wid * NCK, NCK)], didx)

        @pl.when(sid < _NZW)
        def _():
            _zero_rows(zbuf, acc, sid * ROWS_Z, ROWS_Z)

        plsc.subcore_barrier()

        def grp_body(g, c):
            descs = []
            for b in range(GRP):
                descs.append(
                    pltpu.async_copy(
                        ones, acc.at[didx.at[g * GRP + b]], sem, add=True
                    )
                )
            for d in descs:
                d.wait()
            return c

        lax.fori_loop(0, NCK // GRP, grp_body, 0)
        plsc.subcore_barrier()

        @pl.when(sid < _NZW)
        def _():
            _copy_out(acc, out_hbm.at[cid], sid * ROWS_Z, ROWS_Z)

    return deg_kernel


_IB = 16
_JUNK = 128


def _make_prop(N, E_pad, D):
    NCK = E_pad // (_NW * _CH)
    NIB = NCK // _IB
    assert NCK * _NW * _CH == E_pad and NCK % _IB == 0 and _IB % 2 == 0

    ROWS_Z = N // _NZW
    assert ROWS_Z * _NZW == N and ROWS_Z % 8 == 0

    @functools.partial(
        pl.kernel,
        out_type=jax.ShapeDtypeStruct((_NC, N, D), jnp.float32),
        mesh=_MESH,
        scratch_types=[
            [pltpu.VMEM((_IB, _CH), jnp.int32) for _ in range(2)],
            [pltpu.VMEM((_IB, _CH), jnp.int32) for _ in range(2)],
            pltpu.VMEM((_CH, D), jnp.float32),
            pltpu.VMEM_SHARED((N + _JUNK, D), jnp.float32),
            pltpu.SemaphoreType.DMA,
            pltpu.SemaphoreType.DMA,
            pltpu.SemaphoreType.DMA,
        ],
    )
    def prop_kernel(
        u_hbm, src_hbm, dst_hbm, out_hbm,
        sidx, didx, rows, acc, sem_i, sem_g, sem_s,
    ):
        cid = lax.axis_index("c")
        sid = lax.axis_index("s")
        wid = cid * _NS + sid
        row0_w = wid * NCK
        z16 = jnp.zeros((16,), jnp.float32)

        def zinit(i, c):
            for j in range(D // 16):
                rows[i, pl.ds(16 * j, 16)] = z16
            return c

        lax.fori_loop(0, _CH, zinit, 0)

        @pl.when(sid < _NZW)
        def _():
            _zero_rows(rows, acc, sid * ROWS_Z, ROWS_Z)

        pltpu.sync_copy(src_hbm.at[pl.ds(row0_w, _IB)], sidx[0])
        pltpu.sync_copy(dst_hbm.at[pl.ds(row0_w, _IB)], didx[0])
        plsc.subcore_barrier()

        def make_chunk(sb, db):
            def chunk(q, c):
                pltpu.async_copy(u_hbm.at[sb.at[q]], rows, sem_g).wait()
                pltpu.sync_copy(rows, acc.at[db.at[q]], add=True)
                return c

            return chunk

        refill = None
        for blk in range(NIB):
            par = blk % 2
            if blk + 1 < NIB:
                r0 = row0_w + (blk + 1) * _IB
                refill = [
                    pltpu.async_copy(src_hbm.at[pl.ds(r0, _IB)], sidx[1 - par], sem_i),
                    pltpu.async_copy(dst_hbm.at[pl.ds(r0, _IB)], didx[1 - par], sem_i),
                ]
            lax.fori_loop(0, _IB, make_chunk(sidx[par], didx[par]), 0)
            if blk + 1 < NIB:
                for d in refill:
                    d.wait()
        plsc.subcore_barrier()

        @pl.when(sid < _NZW)
        def _():
            _copy_out(acc, out_hbm.at[cid], sid * ROWS_Z, ROWS_Z)

    return prop_kernel


_BLK = 1000


def _tc_grid(N):
    assert N % _BLK == 0
    return N // _BLK


def _dinv_u_kernel(degp_ref, x_ref, dinv_ref, u_ref):
    deg = degp_ref[0, :, 0:1] + degp_ref[1, :, 0:1] + 1.0
    dv = lax.rsqrt(deg)
    dinv_ref[...] = dv
    u_ref[...] = dv * x_ref[...]


def _dinv_and_u(degp, x):
    N, D = x.shape
    return pl.pallas_call(
        _dinv_u_kernel,
        grid=(_tc_grid(N),),
        in_specs=[
            pl.BlockSpec((_NC, _BLK, 16), lambda i: (0, i, 0)),
            pl.BlockSpec((_BLK, D), lambda i: (i, 0)),
        ],
        out_specs=[
            pl.BlockSpec((_BLK, 1), lambda i: (i, 0)),
            pl.BlockSpec((_BLK, D), lambda i: (i, 0)),
        ],
        out_shape=[
            jax.ShapeDtypeStruct((N, 1), jnp.float32),
            jax.ShapeDtypeStruct((N, D), jnp.float32),
        ],
    )(degp, x)


def _matmul_in_kernel(rp_ref, u_ref, dinv_ref, w_ref, b_ref, o_ref):
    a = dinv_ref[...] * (rp_ref[0] + rp_ref[1] + u_ref[...])
    h = jnp.dot(a, w_ref[...], preferred_element_type=jnp.float32)
    o_ref[...] = jnp.maximum(h + b_ref[...], 0.0)


def _prop_matmul_relu(rp, u, dinv, w, b):
    N, D = u.shape
    K = w.shape[1]
    return pl.pallas_call(
        _matmul_in_kernel,
        grid=(_tc_grid(N),),
        in_specs=[
            pl.BlockSpec((_NC, _BLK, D), lambda i: (0, i, 0)),
            pl.BlockSpec((_BLK, D), lambda i: (i, 0)),
            pl.BlockSpec((_BLK, 1), lambda i: (i, 0)),
            pl.BlockSpec((D, K), lambda i: (0, 0)),
            pl.BlockSpec((1, K), lambda i: (0, 0)),
        ],
        out_specs=pl.BlockSpec((_BLK, K), lambda i: (i, 0)),
        out_shape=jax.ShapeDtypeStruct((N, K), jnp.float32),
    )(rp, u, dinv, w, b)


def _matmul_out_kernel(h_ref, w_ref, dinv_ref, o_ref):
    t = jnp.dot(h_ref[...], w_ref[...], preferred_element_type=jnp.float32)
    o_ref[...] = dinv_ref[...] * t


def _matmul_scale(h, w, dinv):
    N, D = h.shape
    K = w.shape[1]
    return pl.pallas_call(
        _matmul_out_kernel,
        grid=(_tc_grid(N),),
        in_specs=[
            pl.BlockSpec((_BLK, D), lambda i: (i, 0)),
            pl.BlockSpec((D, K), lambda i: (0, 0)),
            pl.BlockSpec((_BLK, 1), lambda i: (i, 0)),
        ],
        out_specs=pl.BlockSpec((_BLK, K), lambda i: (i, 0)),
        out_shape=jax.ShapeDtypeStruct((N, K), jnp.float32),
    )(h, w, dinv)


def _combine2_kernel(rp_ref, u_ref, dinv_ref, b_ref, z_ref, un_ref):
    dv = dinv_ref[...]
    z = dv * (rp_ref[0] + rp_ref[1] + u_ref[...]) + b_ref[...]
    z_ref[...] = z
    un_ref[...] = dv * z


def _combine2(rp, u, dinv, b):
    N, D = u.shape
    return pl.pallas_call(
        _combine2_kernel,
        grid=(_tc_grid(N),),
        in_specs=[
            pl.BlockSpec((_NC, _BLK, D), lambda i: (0, i, 0)),
            pl.BlockSpec((_BLK, D), lambda i: (i, 0)),
            pl.BlockSpec((_BLK, 1), lambda i: (i, 0)),
            pl.BlockSpec((1, D), lambda i: (0, 0)),
        ],
        out_specs=[
            pl.BlockSpec((_BLK, D), lambda i: (i, 0)),
            pl.BlockSpec((_BLK, D), lambda i: (i, 0)),
        ],
        out_shape=[
            jax.ShapeDtypeStruct((N, D), jnp.float32),
            jax.ShapeDtypeStruct((N, D), jnp.float32),
        ],
    )(rp, u, dinv, b)


def _combine1_kernel(rp_ref, u_ref, dinv_ref, b_ref, z_ref):
    z_ref[...] = dinv_ref[...] * (rp_ref[0] + rp_ref[1] + u_ref[...]) + b_ref[...]


def _combine1(rp, u, dinv, b):
    N, D = u.shape
    return pl.pallas_call(
        _combine1_kernel,
        grid=(_tc_grid(N),),
        in_specs=[
            pl.BlockSpec((_NC, _BLK, D), lambda i: (0, i, 0)),
            pl.BlockSpec((_BLK, D), lambda i: (i, 0)),
            pl.BlockSpec((_BLK, 1), lambda i: (i, 0)),
            pl.BlockSpec((1, D), lambda i: (0, 0)),
        ],
        out_specs=pl.BlockSpec((_BLK, D), lambda i: (i, 0)),
        out_shape=jax.ShapeDtypeStruct((N, D), jnp.float32),
    )(rp, u, dinv, b)


def kernel(x, edge_index, W1, b1, W2, b2, W3, b3, W4, b4):
    N, D = x.shape
    E = edge_index.shape[1]
    ei = edge_index.astype(jnp.int32)

    quantum = _NW * _CH * _IB
    E_pad = ((E + quantum - 1) // quantum) * quantum
    pad = E_pad - E
    src = jnp.concatenate([ei[0], jnp.zeros((pad,), jnp.int32)])
    dst = jnp.concatenate([ei[1], N + jnp.arange(pad, dtype=jnp.int32) % _JUNK])
    src = src.reshape(E_pad // _CH, _CH)
    dst = dst.reshape(E_pad // _CH, _CH)

    degree = _make_degree(N, E_pad)
    prop = _make_prop(N, E_pad, D)

    degp = degree(dst)
    dinv, u1 = _dinv_and_u(degp, x)

    r1 = prop(u1, src, dst)
    h = _prop_matmul_relu(r1, u1, dinv, W1, b1.reshape(1, -1))
    u2 = _matmul_scale(h, W2, dinv)

    r2 = prop(u2, src, dst)
    z, u3 = _combine2(r2, u2, dinv, b2.reshape(1, -1))

    r3 = prop(u3, src, dst)
    h2 = _prop_matmul_relu(r3, u3, dinv, W3, b3.reshape(1, -1))
    u4 = _matmul_scale(h2, W4, dinv)

    r4 = prop(u4, src, dst)
    recon = _combine1(r4, u4, dinv, b4.reshape(1, -1))
    return (z, recon)


# speedup vs baseline: 2.4642x; 2.4642x over previous
import functools

import jax
import jax.numpy as jnp
from jax import lax
from jax.experimental import pallas as pl
from jax.experimental.pallas import tpu as pltpu
from jax.experimental.pallas import tpu_sc as plsc

_NC = 2
_NS = 16
_NW = _NC * _NS

_MESH = plsc.VectorSubcoreMesh(
    core_axis_name="c", subcore_axis_name="s", num_cores=_NC, num_subcores=_NS
)

_CH = 128

_NZW = 10


def _zero_vmem(buf, n_rows, width):
    z16 = jnp.zeros((16,), jnp.float32)

    def body(i, c):
        for j in range(width // 16):
            buf[i, pl.ds(16 * j, 16)] = z16
        return c

    lax.fori_loop(0, n_rows, body, 0)


def _zero_rows(zstage, acc, row0, n_rows):
    n_full, tail = divmod(n_rows, _CH)
    assert tail % 8 == 0

    def body(k, c):
        pltpu.sync_copy(zstage, acc.at[pl.ds(row0 + k * _CH, _CH)])
        return c

    lax.fori_loop(0, n_full, body, 0)
    if tail:
        pltpu.sync_copy(
            zstage.at[pl.ds(0, tail)], acc.at[pl.ds(row0 + n_full * _CH, tail)]
        )


def _copy_out(acc, out_dst, row0, n_rows):
    pltpu.sync_copy(acc.at[pl.ds(row0, n_rows)], out_dst.at[pl.ds(row0, n_rows)])


def _make_degree(N, E_pad):
    NCK = E_pad // (_NW * _CH)
    assert NCK * _NW * _CH == E_pad and NCK % 8 == 0
    ROWS_Z = N // _NZW
    assert ROWS_Z * _NZW == N and ROWS_Z % 8 == 0
    GRP = 8

    @functools.partial(
        pl.kernel,
        out_type=jax.ShapeDtypeStruct((_NC, N, 16), jnp.float32),
        mesh=_MESH,
        scratch_types=[
            pltpu.VMEM((NCK, _CH), jnp.int32),
            pltpu.VMEM((_CH, 16), jnp.float32),
            pltpu.VMEM((_CH, 16), jnp.float32),
            pltpu.VMEM_SHARED((N + _JUNK, 16), jnp.float32),
            pltpu.SemaphoreType.DMA,
        ],
    )
    def deg_kernel(dst_hbm, out_hbm, didx, ones, zbuf, acc, sem):
        cid = lax.axis_index("c")
        sid = lax.axis_index("s")
        wid = cid * _NS + sid
        one16 = jnp.ones((16,), jnp.float32)
        z16 = jnp.zeros((16,), jnp.float32)

        def init_body(i, c):
            ones[i, pl.ds(0, 16)] = one16
            zbuf[i, pl.ds(0, 16)] = z16
            return c

        lax.fori_loop(0, _CH, init_body, 0)
        pltpu.sync_copy(dst_hbm.at[pl.ds(wid * NCK, NCK)], didx)

        @pl.when(sid < _NZW)
        def _():
            _zero_rows(zbuf, acc, sid * ROWS_Z, ROWS_Z)

        plsc.subcore_barrier()

        def grp_body(g, c):
            descs = []
            for b in range(GRP):
                descs.append(
                    pltpu.async_copy(
                        ones, acc.at[didx.at[g * GRP + b]], sem, add=True
                    )
                )
            for d in descs:
                d.wait()
            return c

        lax.fori_loop(0, NCK // GRP, grp_body, 0)
        plsc.subcore_barrier()

        @pl.when(sid < _NZW)
        def _():
            _copy_out(acc, out_hbm.at[cid], sid * ROWS_Z, ROWS_Z)

    return deg_kernel


_IB = 16
_JUNK = 128


def _make_prop(N, E_pad, D):
    NCK = E_pad // (_NW * _CH)
    NIB = NCK // _IB
    assert NCK * _NW * _CH == E_pad and NCK % _IB == 0 and _IB % 2 == 0

    ROWS_Z = N // _NZW
    assert ROWS_Z * _NZW == N and ROWS_Z % 8 == 0

    @functools.partial(
        pl.kernel,
        out_type=jax.ShapeDtypeStruct((_NC, N, D), jnp.float32),
        mesh=_MESH,
        scratch_types=[
            [pltpu.VMEM((_IB, _CH), jnp.int32) for _ in range(2)],
            [pltpu.VMEM((_IB, _CH), jnp.int32) for _ in range(2)],
            pltpu.VMEM((_CH, D), jnp.float32),
            pltpu.VMEM_SHARED((N + _JUNK, D), jnp.float32),
            pltpu.SemaphoreType.DMA,
            pltpu.SemaphoreType.DMA,
            pltpu.SemaphoreType.DMA,
        ],
    )
    def prop_kernel(
        u_hbm, src_hbm, dst_hbm, out_hbm,
        sidx, didx, rows, acc, sem_i, sem_g, sem_s,
    ):
        cid = lax.axis_index("c")
        sid = lax.axis_index("s")
        wid = cid * _NS + sid
        row0_w = wid * NCK
        z16 = jnp.zeros((16,), jnp.float32)

        def zinit(i, c):
            for j in range(D // 16):
                rows[i, pl.ds(16 * j, 16)] = z16
            return c

        lax.fori_loop(0, _CH, zinit, 0)

        @pl.when(sid < _NZW)
        def _():
            _zero_rows(rows, acc, sid * ROWS_Z, ROWS_Z)

        pltpu.sync_copy(src_hbm.at[pl.ds(row0_w, _IB)], sidx[0])
        pltpu.sync_copy(dst_hbm.at[pl.ds(row0_w, _IB)], didx[0])
        plsc.subcore_barrier()

        def make_chunk(sb, db):
            def chunk(q, c):
                pltpu.async_copy(u_hbm.at[sb.at[q]], rows, sem_g).wait()
                pltpu.sync_copy(rows, acc.at[db.at[q]], add=True)
                return c

            return chunk

        refill = None
        for blk in range(NIB):
            par = blk % 2
            if blk + 1 < NIB:
                r0 = row0_w + (blk + 1) * _IB
                refill = [
                    pltpu.async_copy(src_hbm.at[pl.ds(r0, _IB)], sidx[1 - par], sem_i),
                    pltpu.async_copy(dst_hbm.at[pl.ds(r0, _IB)], didx[1 - par], sem_i),
                ]
            lax.fori_loop(0, _IB, make_chunk(sidx[par], didx[par]), 0)
            if blk + 1 < NIB:
                for d in refill:
                    d.wait()
        plsc.subcore_barrier()

        @pl.when(sid < _NZW)
        def _():
            _copy_out(acc, out_hbm.at[cid], sid * ROWS_Z, ROWS_Z)

    return prop_kernel


_BLK = 1000


def _tc_grid(N):
    assert N % _BLK == 0
    return N // _BLK


def _dinv_u_kernel(degp_ref, x_ref, dinv_ref, u_ref):
    deg = degp_ref[0, :, 0:1] + degp_ref[1, :, 0:1] + 1.0
    dv = lax.rsqrt(deg)
    dinv_ref[...] = dv
    u_ref[...] = dv * x_ref[...]


def _dinv_and_u(degp, x):
    N, D = x.shape
    return pl.pallas_call(
        _dinv_u_kernel,
        grid=(_tc_grid(N),),
        in_specs=[
            pl.BlockSpec((_NC, _BLK, 16), lambda i: (0, i, 0)),
            pl.BlockSpec((_BLK, D), lambda i: (i, 0)),
        ],
        out_specs=[
            pl.BlockSpec((_BLK, 1), lambda i: (i, 0)),
            pl.BlockSpec((_BLK, D), lambda i: (i, 0)),
        ],
        out_shape=[
            jax.ShapeDtypeStruct((N, 1), jnp.float32),
            jax.ShapeDtypeStruct((N, D), jnp.float32),
        ],
    )(degp, x)


def _matmul_in_kernel(rp_ref, u_ref, dinv_ref, w_ref, b_ref, o_ref):
    a = dinv_ref[...] * (rp_ref[0] + rp_ref[1] + u_ref[...])
    h = jnp.dot(a, w_ref[...], preferred_element_type=jnp.float32)
    o_ref[...] = jnp.maximum(h + b_ref[...], 0.0)


def _prop_matmul_relu(rp, u, dinv, w, b):
    N, D = u.shape
    K = w.shape[1]
    return pl.pallas_call(
        _matmul_in_kernel,
        grid=(_tc_grid(N),),
        in_specs=[
            pl.BlockSpec((_NC, _BLK, D), lambda i: (0, i, 0)),
            pl.BlockSpec((_BLK, D), lambda i: (i, 0)),
            pl.BlockSpec((_BLK, 1), lambda i: (i, 0)),
            pl.BlockSpec((D, K), lambda i: (0, 0)),
            pl.BlockSpec((1, K), lambda i: (0, 0)),
        ],
        out_specs=pl.BlockSpec((_BLK, K), lambda i: (i, 0)),
        out_shape=jax.ShapeDtypeStruct((N, K), jnp.float32),
    )(rp, u, dinv, w, b)


def _matmul_out_kernel(h_ref, w_ref, dinv_ref, o_ref):
    t = jnp.dot(h_ref[...], w_ref[...], preferred_element_type=jnp.float32)
    o_ref[...] = dinv_ref[...] * t


def _matmul_scale(h, w, dinv):
    N, D = h.shape
    K = w.shape[1]
    return pl.pallas_call(
        _matmul_out_kernel,
        grid=(_tc_grid(N),),
        in_specs=[
            pl.BlockSpec((_BLK, D), lambda i: (i, 0)),
            pl.BlockSpec((D, K), lambda i: (0, 0)),
            pl.BlockSpec((_BLK, 1), lambda i: (i, 0)),
        ],
        out_specs=pl.BlockSpec((_BLK, K), lambda i: (i, 0)),
        out_shape=jax.ShapeDtypeStruct((N, K), jnp.float32),
    )(h, w, dinv)


def _combine2_kernel(rp_ref, u_ref, dinv_ref, b_ref, z_ref, un_ref):
    dv = dinv_ref[...]
    z = dv * (rp_ref[0] + rp_ref[1] + u_ref[...]) + b_ref[...]
    z_ref[...] = z
    un_ref[...] = dv * z


def _combine2(rp, u, dinv, b):
    N, D = u.shape
    return pl.pallas_call(
        _combine2_kernel,
        grid=(_tc_grid(N),),
        in_specs=[
            pl.BlockSpec((_NC, _BLK, D), lambda i: (0, i, 0)),
            pl.BlockSpec((_BLK, D), lambda i: (i, 0)),
            pl.BlockSpec((_BLK, 1), lambda i: (i, 0)),
            pl.BlockSpec((1, D), lambda i: (0, 0)),
        ],
        out_specs=[
            pl.BlockSpec((_BLK, D), lambda i: (i, 0)),
            pl.BlockSpec((_BLK, D), lambda i: (i, 0)),
        ],
        out_shape=[
            jax.ShapeDtypeStruct((N, D), jnp.float32),
            jax.ShapeDtypeStruct((N, D), jnp.float32),
        ],
    )(rp, u, dinv, b)


def _combine1_kernel(rp_ref, u_ref, dinv_ref, b_ref, z_ref):
    z_ref[...] = dinv_ref[...] * (rp_ref[0] + rp_ref[1] + u_ref[...]) + b_ref[...]


def _combine1(rp, u, dinv, b):
    N, D = u.shape
    return pl.pallas_call(
        _combine1_kernel,
        grid=(_tc_grid(N),),
        in_specs=[
            pl.BlockSpec((_NC, _BLK, D), lambda i: (0, i, 0)),
            pl.BlockSpec((_BLK, D), lambda i: (i, 0)),
            pl.BlockSpec((_BLK, 1), lambda i: (i, 0)),
            pl.BlockSpec((1, D), lambda i: (0, 0)),
        ],
        out_specs=pl.BlockSpec((_BLK, D), lambda i: (i, 0)),
        out_shape=jax.ShapeDtypeStruct((N, D), jnp.float32),
    )(rp, u, dinv, b)


def kernel(x, edge_index, W1, b1, W2, b2, W3, b3, W4, b4):
    N, D = x.shape
    E = edge_index.shape[1]
    ei = edge_index.astype(jnp.int32)

    quantum = _NW * _CH * _IB
    E_pad = ((E + quantum - 1) // quantum) * quantum
    pad = E_pad - E
    src = jnp.concatenate([ei[0], jnp.arange(pad, dtype=jnp.int32) % N])
    dst = jnp.concatenate([ei[1], N + jnp.arange(pad, dtype=jnp.int32) % _JUNK])
    src = src.reshape(E_pad // _CH, _CH)
    dst = dst.reshape(E_pad // _CH, _CH)

    degree = _make_degree(N, E_pad)
    prop = _make_prop(N, E_pad, D)

    degp = degree(dst)
    dinv, u1 = _dinv_and_u(degp, x)

    r1 = prop(u1, src, dst)
    h = _prop_matmul_relu(r1, u1, dinv, W1, b1.reshape(1, -1))
    u2 = _matmul_scale(h, W2, dinv)

    r2 = prop(u2, src, dst)
    z, u3 = _combine2(r2, u2, dinv, b2.reshape(1, -1))

    r3 = prop(u3, src, dst)
    h2 = _prop_matmul_relu(r3, u3, dinv, W3, b3.reshape(1, -1))
    u4 = _matmul_scale(h2, W4, dinv)

    r4 = prop(u4, src, dst)
    recon = _combine1(r4, u4, dinv, b4.reshape(1, -1))
    return (z, recon)


# speedup vs baseline: 3.1615x; 1.2830x over previous
import functools

import jax
import jax.numpy as jnp
from jax import lax
from jax.experimental import pallas as pl
from jax.experimental.pallas import tpu as pltpu
from jax.experimental.pallas import tpu_sc as plsc

_NC = 2
_NS = 16
_NW = _NC * _NS

_MESH = plsc.VectorSubcoreMesh(
    core_axis_name="c", subcore_axis_name="s", num_cores=_NC, num_subcores=_NS
)

_CH = 128

_NZW = 10


def _zero_vmem(buf, n_rows, width):
    z16 = jnp.zeros((16,), jnp.float32)

    def body(i, c):
        for j in range(width // 16):
            buf[i, pl.ds(16 * j, 16)] = z16
        return c

    lax.fori_loop(0, n_rows, body, 0)


def _zero_rows(zstage, acc, row0, n_rows):
    n_full, tail = divmod(n_rows, _CH)
    assert tail % 8 == 0

    def body(k, c):
        pltpu.sync_copy(zstage, acc.at[pl.ds(row0 + k * _CH, _CH)])
        return c

    lax.fori_loop(0, n_full, body, 0)
    if tail:
        pltpu.sync_copy(
            zstage.at[pl.ds(0, tail)], acc.at[pl.ds(row0 + n_full * _CH, tail)]
        )


def _copy_out(acc, out_dst, row0, n_rows):
    pltpu.sync_copy(acc.at[pl.ds(row0, n_rows)], out_dst.at[pl.ds(row0, n_rows)])


def _make_degree(N, E_pad):
    NCK = E_pad // (_NW * _CH)
    assert NCK * _NW * _CH == E_pad and NCK % 8 == 0
    ROWS_Z = N // _NZW
    assert ROWS_Z * _NZW == N and ROWS_Z % 8 == 0
    GRP = 8

    @functools.partial(
        pl.kernel,
        out_type=jax.ShapeDtypeStruct((_NC, N, 16), jnp.float32),
        mesh=_MESH,
        scratch_types=[
            pltpu.VMEM((NCK, _CH), jnp.int32),
            pltpu.VMEM((_CH, 16), jnp.float32),
            pltpu.VMEM((_CH, 16), jnp.float32),
            pltpu.VMEM_SHARED((N + _JUNK, 16), jnp.float32),
            pltpu.SemaphoreType.DMA,
        ],
    )
    def deg_kernel(dst_hbm, out_hbm, didx, ones, zbuf, acc, sem):
        cid = lax.axis_index("c")
        sid = lax.axis_index("s")
        wid = cid * _NS + sid
        one16 = jnp.ones((16,), jnp.float32)
        z16 = jnp.zeros((16,), jnp.float32)

        def init_body(i, c):
            ones[i, pl.ds(0, 16)] = one16
            zbuf[i, pl.ds(0, 16)] = z16
            return c

        lax.fori_loop(0, _CH, init_body, 0)
        pltpu.sync_copy(dst_hbm.at[pl.ds(wid * NCK, NCK)], didx)

        @pl.when(sid < _NZW)
        def _():
            _zero_rows(zbuf, acc, sid * ROWS_Z, ROWS_Z)

        plsc.subcore_barrier()

        def grp_body(g, c):
            descs = []
            for b in range(GRP):
                descs.append(
                    pltpu.async_copy(
                        ones, acc.at[didx.at[g * GRP + b]], sem, add=True
                    )
                )
            for d in descs:
                d.wait()
            return c

        lax.fori_loop(0, NCK // GRP, grp_body, 0)
        plsc.subcore_barrier()

        @pl.when(sid < _NZW)
        def _():
            _copy_out(acc, out_hbm.at[cid], sid * ROWS_Z, ROWS_Z)

    return deg_kernel


_IB = 16
_JUNK = 128


def _make_prop(N, E_pad, D):
    NCK = E_pad // (_NW * _CH)
    NIB = NCK // _IB
    assert NCK * _NW * _CH == E_pad and NCK % _IB == 0 and _IB % 2 == 0

    ROWS_Z = N // _NZW
    assert ROWS_Z * _NZW == N and ROWS_Z % 8 == 0

    @functools.partial(
        pl.kernel,
        out_type=jax.ShapeDtypeStruct((_NC, N, D), jnp.float32),
        mesh=_MESH,
        scratch_types=[
            [pltpu.VMEM((_IB, _CH), jnp.int32) for _ in range(2)],
            [pltpu.VMEM((_IB, _CH), jnp.int32) for _ in range(2)],
            [pltpu.VMEM((_CH, D), jnp.float32) for _ in range(2)],
            pltpu.VMEM_SHARED((N + _JUNK, D), jnp.float32),
            pltpu.SemaphoreType.DMA,
            pltpu.SemaphoreType.DMA,
            pltpu.SemaphoreType.DMA,
        ],
    )
    def prop_kernel(
        u_hbm, src_hbm, dst_hbm, out_hbm,
        sidx, didx, rows, acc, sem_i, sem_g, sem_s,
    ):
        cid = lax.axis_index("c")
        sid = lax.axis_index("s")
        wid = cid * _NS + sid
        row0_w = wid * NCK
        z16 = jnp.zeros((16,), jnp.float32)

        def zinit(i, c):
            for j in range(D // 16):
                rows[0][i, pl.ds(16 * j, 16)] = z16
            return c

        lax.fori_loop(0, _CH, zinit, 0)

        @pl.when(sid < _NZW)
        def _():
            _zero_rows(rows[0], acc, sid * ROWS_Z, ROWS_Z)

        pltpu.sync_copy(src_hbm.at[pl.ds(row0_w, _IB)], sidx[0])
        pltpu.sync_copy(dst_hbm.at[pl.ds(row0_w, _IB)], didx[0])
        g0 = [
            pltpu.async_copy(u_hbm.at[sidx[0].at[b]], rows[b], sem_g)
            for b in range(2)
        ]
        plsc.subcore_barrier()

        def drain_scatter(buf):
            pltpu.make_async_copy(u_hbm.at[pl.ds(0, _CH)], buf, sem_s).wait()

        for b in range(2):
            g0[b].wait()
            pltpu.async_copy(rows[b], acc.at[didx[0].at[b]], sem_s, add=True)

        def make_pair(sb, db):
            def pair(q, c):
                for b in range(2):
                    drain_scatter(rows[b])
                    gd = pltpu.async_copy(u_hbm.at[sb.at[2 * q + b]], rows[b], sem_g)
                    gd.wait()
                    pltpu.async_copy(
                        rows[b], acc.at[db.at[2 * q + b]], sem_s, add=True
                    )
                return c

            return pair

        refill = None
        for blk in range(NIB):
            par = blk % 2
            if blk + 1 < NIB:
                r0 = row0_w + (blk + 1) * _IB
                refill = [
                    pltpu.async_copy(src_hbm.at[pl.ds(r0, _IB)], sidx[1 - par], sem_i),
                    pltpu.async_copy(dst_hbm.at[pl.ds(r0, _IB)], didx[1 - par], sem_i),
                ]
            q0 = 1 if blk == 0 else 0
            lax.fori_loop(q0, _IB // 2, make_pair(sidx[par], didx[par]), 0)
            if blk + 1 < NIB:
                for d in refill:
                    d.wait()
        for b in range(2):
            drain_scatter(rows[b])
        plsc.subcore_barrier()

        @pl.when(sid < _NZW)
        def _():
            _copy_out(acc, out_hbm.at[cid], sid * ROWS_Z, ROWS_Z)

    return prop_kernel


_BLK = 1000


def _tc_grid(N):
    assert N % _BLK == 0
    return N // _BLK


def _dinv_u_kernel(degp_ref, x_ref, dinv_ref, u_ref):
    deg = degp_ref[0, :, 0:1] + degp_ref[1, :, 0:1] + 1.0
    dv = lax.rsqrt(deg)
    dinv_ref[...] = dv
    u_ref[...] = dv * x_ref[...]


def _dinv_and_u(degp, x):
    N, D = x.shape
    return pl.pallas_call(
        _dinv_u_kernel,
        grid=(_tc_grid(N),),
        in_specs=[
            pl.BlockSpec((_NC, _BLK, 16), lambda i: (0, i, 0)),
            pl.BlockSpec((_BLK, D), lambda i: (i, 0)),
        ],
        out_specs=[
            pl.BlockSpec((_BLK, 1), lambda i: (i, 0)),
            pl.BlockSpec((_BLK, D), lambda i: (i, 0)),
        ],
        out_shape=[
            jax.ShapeDtypeStruct((N, 1), jnp.float32),
            jax.ShapeDtypeStruct((N, D), jnp.float32),
        ],
    )(degp, x)


def _matmul_in_kernel(rp_ref, u_ref, dinv_ref, w_ref, b_ref, o_ref):
    a = dinv_ref[...] * (rp_ref[0] + rp_ref[1] + u_ref[...])
    h = jnp.dot(a, w_ref[...], preferred_element_type=jnp.float32)
    o_ref[...] = jnp.maximum(h + b_ref[...], 0.0)


def _prop_matmul_relu(rp, u, dinv, w, b):
    N, D = u.shape
    K = w.shape[1]
    return pl.pallas_call(
        _matmul_in_kernel,
        grid=(_tc_grid(N),),
        in_specs=[
            pl.BlockSpec((_NC, _BLK, D), lambda i: (0, i, 0)),
            pl.BlockSpec((_BLK, D), lambda i: (i, 0)),
            pl.BlockSpec((_BLK, 1), lambda i: (i, 0)),
            pl.BlockSpec((D, K), lambda i: (0, 0)),
            pl.BlockSpec((1, K), lambda i: (0, 0)),
        ],
        out_specs=pl.BlockSpec((_BLK, K), lambda i: (i, 0)),
        out_shape=jax.ShapeDtypeStruct((N, K), jnp.float32),
    )(rp, u, dinv, w, b)


def _matmul_out_kernel(h_ref, w_ref, dinv_ref, o_ref):
    t = jnp.dot(h_ref[...], w_ref[...], preferred_element_type=jnp.float32)
    o_ref[...] = dinv_ref[...] * t


def _matmul_scale(h, w, dinv):
    N, D = h.shape
    K = w.shape[1]
    return pl.pallas_call(
        _matmul_out_kernel,
        grid=(_tc_grid(N),),
        in_specs=[
            pl.BlockSpec((_BLK, D), lambda i: (i, 0)),
            pl.BlockSpec((D, K), lambda i: (0, 0)),
            pl.BlockSpec((_BLK, 1), lambda i: (i, 0)),
        ],
        out_specs=pl.BlockSpec((_BLK, K), lambda i: (i, 0)),
        out_shape=jax.ShapeDtypeStruct((N, K), jnp.float32),
    )(h, w, dinv)


def _combine2_kernel(rp_ref, u_ref, dinv_ref, b_ref, z_ref, un_ref):
    dv = dinv_ref[...]
    z = dv * (rp_ref[0] + rp_ref[1] + u_ref[...]) + b_ref[...]
    z_ref[...] = z
    un_ref[...] = dv * z


def _combine2(rp, u, dinv, b):
    N, D = u.shape
    return pl.pallas_call(
        _combine2_kernel,
        grid=(_tc_grid(N),),
        in_specs=[
            pl.BlockSpec((_NC, _BLK, D), lambda i: (0, i, 0)),
            pl.BlockSpec((_BLK, D), lambda i: (i, 0)),
            pl.BlockSpec((_BLK, 1), lambda i: (i, 0)),
            pl.BlockSpec((1, D), lambda i: (0, 0)),
        ],
        out_specs=[
            pl.BlockSpec((_BLK, D), lambda i: (i, 0)),
            pl.BlockSpec((_BLK, D), lambda i: (i, 0)),
        ],
        out_shape=[
            jax.ShapeDtypeStruct((N, D), jnp.float32),
            jax.ShapeDtypeStruct((N, D), jnp.float32),
        ],
    )(rp, u, dinv, b)


def _combine1_kernel(rp_ref, u_ref, dinv_ref, b_ref, z_ref):
    z_ref[...] = dinv_ref[...] * (rp_ref[0] + rp_ref[1] + u_ref[...]) + b_ref[...]


def _combine1(rp, u, dinv, b):
    N, D = u.shape
    return pl.pallas_call(
        _combine1_kernel,
        grid=(_tc_grid(N),),
        in_specs=[
            pl.BlockSpec((_NC, _BLK, D), lambda i: (0, i, 0)),
            pl.BlockSpec((_BLK, D), lambda i: (i, 0)),
            pl.BlockSpec((_BLK, 1), lambda i: (i, 0)),
            pl.BlockSpec((1, D), lambda i: (0, 0)),
        ],
        out_specs=pl.BlockSpec((_BLK, D), lambda i: (i, 0)),
        out_shape=jax.ShapeDtypeStruct((N, D), jnp.float32),
    )(rp, u, dinv, b)


def kernel(x, edge_index, W1, b1, W2, b2, W3, b3, W4, b4):
    N, D = x.shape
    E = edge_index.shape[1]
    ei = edge_index.astype(jnp.int32)

    quantum = _NW * _CH * _IB
    E_pad = ((E + quantum - 1) // quantum) * quantum
    pad = E_pad - E
    src = jnp.concatenate([ei[0], jnp.arange(pad, dtype=jnp.int32) % N])
    dst = jnp.concatenate([ei[1], N + jnp.arange(pad, dtype=jnp.int32) % _JUNK])
    src = src.reshape(E_pad // _CH, _CH)
    dst = dst.reshape(E_pad // _CH, _CH)

    degree = _make_degree(N, E_pad)
    prop = _make_prop(N, E_pad, D)

    degp = degree(dst)
    dinv, u1 = _dinv_and_u(degp, x)

    r1 = prop(u1, src, dst)
    h = _prop_matmul_relu(r1, u1, dinv, W1, b1.reshape(1, -1))
    u2 = _matmul_scale(h, W2, dinv)

    r2 = prop(u2, src, dst)
    z, u3 = _combine2(r2, u2, dinv, b2.reshape(1, -1))

    r3 = prop(u3, src, dst)
    h2 = _prop_matmul_relu(r3, u3, dinv, W3, b3.reshape(1, -1))
    u4 = _matmul_scale(h2, W4, dinv)

    r4 = prop(u4, src, dst)
    recon = _combine1(r4, u4, dinv, b4.reshape(1, -1))
    return (z, recon)


# speedup vs baseline: 3.1755x; 1.0044x over previous
import functools

import jax
import jax.numpy as jnp
from jax import lax
from jax.experimental import pallas as pl
from jax.experimental.pallas import tpu as pltpu
from jax.experimental.pallas import tpu_sc as plsc

_NC = 2
_NS = 16
_NW = _NC * _NS

_MESH = plsc.VectorSubcoreMesh(
    core_axis_name="c", subcore_axis_name="s", num_cores=_NC, num_subcores=_NS
)

_CH = 128

_NZW = 10


def _zero_vmem(buf, n_rows, width):
    z16 = jnp.zeros((16,), jnp.float32)

    def body(i, c):
        for j in range(width // 16):
            buf[i, pl.ds(16 * j, 16)] = z16
        return c

    lax.fori_loop(0, n_rows, body, 0)


def _zero_rows(zstage, acc, row0, n_rows):
    n_full, tail = divmod(n_rows, _CH)
    assert tail % 8 == 0

    def body(k, c):
        pltpu.sync_copy(zstage, acc.at[pl.ds(row0 + k * _CH, _CH)])
        return c

    lax.fori_loop(0, n_full, body, 0)
    if tail:
        pltpu.sync_copy(
            zstage.at[pl.ds(0, tail)], acc.at[pl.ds(row0 + n_full * _CH, tail)]
        )


def _copy_out(acc, out_dst, row0, n_rows):
    pltpu.sync_copy(acc.at[pl.ds(row0, n_rows)], out_dst.at[pl.ds(row0, n_rows)])


def _make_degree(N, E_pad):
    NCK = E_pad // (_NW * _CH)
    assert NCK * _NW * _CH == E_pad and NCK % 8 == 0
    ROWS_Z = N // _NZW
    assert ROWS_Z * _NZW == N and ROWS_Z % 8 == 0
    GRP = 8

    @functools.partial(
        pl.kernel,
        out_type=jax.ShapeDtypeStruct((_NC, N, 16), jnp.float32),
        mesh=_MESH,
        scratch_types=[
            pltpu.VMEM((NCK, _CH), jnp.int32),
            pltpu.VMEM((_CH, 16), jnp.float32),
            pltpu.VMEM((_CH, 16), jnp.float32),
            pltpu.VMEM_SHARED((N + _JUNK, 16), jnp.float32),
            pltpu.SemaphoreType.DMA,
        ],
    )
    def deg_kernel(dst_hbm, out_hbm, didx, ones, zbuf, acc, sem):
        cid = lax.axis_index("c")
        sid = lax.axis_index("s")
        wid = cid * _NS + sid
        one16 = jnp.ones((16,), jnp.float32)
        z16 = jnp.zeros((16,), jnp.float32)

        def init_body(i, c):
            ones[i, pl.ds(0, 16)] = one16
            zbuf[i, pl.ds(0, 16)] = z16
            return c

        lax.fori_loop(0, _CH, init_body, 0)
        pltpu.sync_copy(dst_hbm.at[pl.ds(wid * NCK, NCK)], didx)

        @pl.when(sid < _NZW)
        def _():
            _zero_rows(zbuf, acc, sid * ROWS_Z, ROWS_Z)

        plsc.subcore_barrier()

        def grp_body(g, c):
            descs = []
            for b in range(GRP):
                descs.append(
                    pltpu.async_copy(
                        ones, acc.at[didx.at[g * GRP + b]], sem, add=True
                    )
                )
            for d in descs:
                d.wait()
            return c

        lax.fori_loop(0, NCK // GRP, grp_body, 0)
        plsc.subcore_barrier()

        @pl.when(sid < _NZW)
        def _():
            _copy_out(acc, out_hbm.at[cid], sid * ROWS_Z, ROWS_Z)

    return deg_kernel


_IB = 16
_JUNK = 128


def _make_prop(N, E_pad, D):
    NCK = E_pad // (_NW * _CH)
    NIB = NCK // _IB
    assert NCK * _NW * _CH == E_pad and NCK % _IB == 0 and _IB % 2 == 0

    ROWS_Z = N // _NZW
    assert ROWS_Z * _NZW == N and ROWS_Z % 8 == 0

    @functools.partial(
        pl.kernel,
        out_type=jax.ShapeDtypeStruct((_NC, N, D), jnp.float32),
        mesh=_MESH,
        scratch_types=[
            [pltpu.VMEM((_IB, _CH), jnp.int32) for _ in range(2)],
            [pltpu.VMEM((_IB, _CH), jnp.int32) for _ in range(2)],
            [pltpu.VMEM((_CH, D), jnp.float32) for _ in range(2)],
            pltpu.VMEM_SHARED((N + _JUNK, D), jnp.float32),
            pltpu.SemaphoreType.DMA,
            pltpu.SemaphoreType.DMA,
            pltpu.SemaphoreType.DMA,
        ],
    )
    def prop_kernel(
        u_hbm, src_hbm, dst_hbm, out_hbm,
        sidx, didx, rows, acc, sem_i, sem_g, sem_s,
    ):
        cid = lax.axis_index("c")
        sid = lax.axis_index("s")
        wid = cid * _NS + sid
        row0_w = wid * NCK
        z16 = jnp.zeros((16,), jnp.float32)

        def zinit(i, c):
            for j in range(D // 16):
                rows[0][i, pl.ds(16 * j, 16)] = z16
            return c

        lax.fori_loop(0, _CH, zinit, 0)

        @pl.when(sid < _NZW)
        def _():
            _zero_rows(rows[0], acc, sid * ROWS_Z, ROWS_Z)

        pltpu.sync_copy(src_hbm.at[pl.ds(row0_w, _IB)], sidx[0])
        pltpu.sync_copy(dst_hbm.at[pl.ds(row0_w, _IB)], didx[0])
        g0 = [
            pltpu.async_copy(u_hbm.at[sidx[0].at[b]], rows[b], sem_g)
            for b in range(2)
        ]
        plsc.subcore_barrier()

        def drain_scatter(buf):
            pltpu.make_async_copy(u_hbm.at[pl.ds(0, _CH)], buf, sem_s).wait()


        g0[0].wait()
        pltpu.async_copy(rows[0], acc.at[didx[0].at[0]], sem_s, add=True)
        drain_scatter(rows[0])
        g0[1].wait()
        pltpu.async_copy(rows[1], acc.at[didx[0].at[1]], sem_s, add=True)

        def make_pair(sb, db):
            def pair(q, c):
                for b in range(2):
                    gd = pltpu.async_copy(u_hbm.at[sb.at[2 * q + b]], rows[b], sem_g)
                    drain_scatter(rows[b])
                    gd.wait()
                    pltpu.async_copy(
                        rows[b], acc.at[db.at[2 * q + b]], sem_s, add=True
                    )
                return c

            return pair

        refill = None
        for blk in range(NIB):
            par = blk % 2
            if blk + 1 < NIB:
                r0 = row0_w + (blk + 1) * _IB
                refill = [
                    pltpu.async_copy(src_hbm.at[pl.ds(r0, _IB)], sidx[1 - par], sem_i),
                    pltpu.async_copy(dst_hbm.at[pl.ds(r0, _IB)], didx[1 - par], sem_i),
                ]
            q0 = 1 if blk == 0 else 0
            lax.fori_loop(q0, _IB // 2, make_pair(sidx[par], didx[par]), 0)
            if blk + 1 < NIB:
                for d in refill:
                    d.wait()
        drain_scatter(rows[1])
        plsc.subcore_barrier()

        @pl.when(sid < _NZW)
        def _():
            _copy_out(acc, out_hbm.at[cid], sid * ROWS_Z, ROWS_Z)

    return prop_kernel


_BLK = 1000


def _tc_grid(N):
    assert N % _BLK == 0
    return N // _BLK


def _dinv_u_kernel(degp_ref, x_ref, dinv_ref, u_ref):
    deg = degp_ref[0, :, 0:1] + degp_ref[1, :, 0:1] + 1.0
    dv = lax.rsqrt(deg)
    dinv_ref[...] = dv
    u_ref[...] = dv * x_ref[...]


def _dinv_and_u(degp, x):
    N, D = x.shape
    return pl.pallas_call(
        _dinv_u_kernel,
        grid=(_tc_grid(N),),
        in_specs=[
            pl.BlockSpec((_NC, _BLK, 16), lambda i: (0, i, 0)),
            pl.BlockSpec((_BLK, D), lambda i: (i, 0)),
        ],
        out_specs=[
            pl.BlockSpec((_BLK, 1), lambda i: (i, 0)),
            pl.BlockSpec((_BLK, D), lambda i: (i, 0)),
        ],
        out_shape=[
            jax.ShapeDtypeStruct((N, 1), jnp.float32),
            jax.ShapeDtypeStruct((N, D), jnp.float32),
        ],
    )(degp, x)


def _matmul_in_kernel(rp_ref, u_ref, dinv_ref, w_ref, b_ref, o_ref):
    a = dinv_ref[...] * (rp_ref[0] + rp_ref[1] + u_ref[...])
    h = jnp.dot(a, w_ref[...], preferred_element_type=jnp.float32)
    o_ref[...] = jnp.maximum(h + b_ref[...], 0.0)


def _prop_matmul_relu(rp, u, dinv, w, b):
    N, D = u.shape
    K = w.shape[1]
    return pl.pallas_call(
        _matmul_in_kernel,
        grid=(_tc_grid(N),),
        in_specs=[
            pl.BlockSpec((_NC, _BLK, D), lambda i: (0, i, 0)),
            pl.BlockSpec((_BLK, D), lambda i: (i, 0)),
            pl.BlockSpec((_BLK, 1), lambda i: (i, 0)),
            pl.BlockSpec((D, K), lambda i: (0, 0)),
            pl.BlockSpec((1, K), lambda i: (0, 0)),
        ],
        out_specs=pl.BlockSpec((_BLK, K), lambda i: (i, 0)),
        out_shape=jax.ShapeDtypeStruct((N, K), jnp.float32),
    )(rp, u, dinv, w, b)


def _matmul_out_kernel(h_ref, w_ref, dinv_ref, o_ref):
    t = jnp.dot(h_ref[...], w_ref[...], preferred_element_type=jnp.float32)
    o_ref[...] = dinv_ref[...] * t


def _matmul_scale(h, w, dinv):
    N, D = h.shape
    K = w.shape[1]
    return pl.pallas_call(
        _matmul_out_kernel,
        grid=(_tc_grid(N),),
        in_specs=[
            pl.BlockSpec((_BLK, D), lambda i: (i, 0)),
            pl.BlockSpec((D, K), lambda i: (0, 0)),
            pl.BlockSpec((_BLK, 1), lambda i: (i, 0)),
        ],
        out_specs=pl.BlockSpec((_BLK, K), lambda i: (i, 0)),
        out_shape=jax.ShapeDtypeStruct((N, K), jnp.float32),
    )(h, w, dinv)


def _combine2_kernel(rp_ref, u_ref, dinv_ref, b_ref, z_ref, un_ref):
    dv = dinv_ref[...]
    z = dv * (rp_ref[0] + rp_ref[1] + u_ref[...]) + b_ref[...]
    z_ref[...] = z
    un_ref[...] = dv * z


def _combine2(rp, u, dinv, b):
    N, D = u.shape
    return pl.pallas_call(
        _combine2_kernel,
        grid=(_tc_grid(N),),
        in_specs=[
            pl.BlockSpec((_NC, _BLK, D), lambda i: (0, i, 0)),
            pl.BlockSpec((_BLK, D), lambda i: (i, 0)),
            pl.BlockSpec((_BLK, 1), lambda i: (i, 0)),
            pl.BlockSpec((1, D), lambda i: (0, 0)),
        ],
        out_specs=[
            pl.BlockSpec((_BLK, D), lambda i: (i, 0)),
            pl.BlockSpec((_BLK, D), lambda i: (i, 0)),
        ],
        out_shape=[
            jax.ShapeDtypeStruct((N, D), jnp.float32),
            jax.ShapeDtypeStruct((N, D), jnp.float32),
        ],
    )(rp, u, dinv, b)


def _combine1_kernel(rp_ref, u_ref, dinv_ref, b_ref, z_ref):
    z_ref[...] = dinv_ref[...] * (rp_ref[0] + rp_ref[1] + u_ref[...]) + b_ref[...]


def _combine1(rp, u, dinv, b):
    N, D = u.shape
    return pl.pallas_call(
        _combine1_kernel,
        grid=(_tc_grid(N),),
        in_specs=[
            pl.BlockSpec((_NC, _BLK, D), lambda i: (0, i, 0)),
            pl.BlockSpec((_BLK, D), lambda i: (i, 0)),
            pl.BlockSpec((_BLK, 1), lambda i: (i, 0)),
            pl.BlockSpec((1, D), lambda i: (0, 0)),
        ],
        out_specs=pl.BlockSpec((_BLK, D), lambda i: (i, 0)),
        out_shape=jax.ShapeDtypeStruct((N, D), jnp.float32),
    )(rp, u, dinv, b)


def kernel(x, edge_index, W1, b1, W2, b2, W3, b3, W4, b4):
    N, D = x.shape
    E = edge_index.shape[1]
    ei = edge_index.astype(jnp.int32)

    quantum = _NW * _CH * _IB
    E_pad = ((E + quantum - 1) // quantum) * quantum
    pad = E_pad - E
    src = jnp.concatenate([ei[0], jnp.arange(pad, dtype=jnp.int32) % N])
    dst = jnp.concatenate([ei[1], N + jnp.arange(pad, dtype=jnp.int32) % _JUNK])
    src = src.reshape(E_pad // _CH, _CH)
    dst = dst.reshape(E_pad // _CH, _CH)

    degree = _make_degree(N, E_pad)
    prop = _make_prop(N, E_pad, D)

    degp = degree(dst)
    dinv, u1 = _dinv_and_u(degp, x)

    r1 = prop(u1, src, dst)
    h = _prop_matmul_relu(r1, u1, dinv, W1, b1.reshape(1, -1))
    u2 = _matmul_scale(h, W2, dinv)

    r2 = prop(u2, src, dst)
    z, u3 = _combine2(r2, u2, dinv, b2.reshape(1, -1))

    r3 = prop(u3, src, dst)
    h2 = _prop_matmul_relu(r3, u3, dinv, W3, b3.reshape(1, -1))
    u4 = _matmul_scale(h2, W4, dinv)

    r4 = prop(u4, src, dst)
    recon = _combine1(r4, u4, dinv, b4.reshape(1, -1))
    return (z, recon)
